# baseline (speedup 1.0000x reference)
import functools

import jax
import jax.numpy as jnp
from jax import lax
from jax.experimental import pallas as pl
from jax.experimental.pallas import tpu as pltpu
from jax.experimental.pallas import tpu_sc as plsc

_NC = 2
_NS = 16
_NW = _NC * _NS
_L = 16


@functools.lru_cache(maxsize=None)
def _build(B, NUMF, CATF, V, D):
    NTOK = 1 + NUMF + CATF
    BPW = B // _NW
    SUB = 16
    NSUB = BPW // SUB
    ND = D // _L

    mesh = plsc.VectorSubcoreMesh(core_axis_name="c", subcore_axis_name="s")

    @functools.partial(
        pl.kernel,
        out_type=jax.ShapeDtypeStruct((B, NTOK, D), jnp.float32),
        mesh=mesh,
        compiler_params=pltpu.CompilerParams(use_tc_tiling_on_sc=False),
        scratch_types=[
            pltpu.VMEM((NUMF, D), jnp.float32),
            pltpu.VMEM((NUMF, D), jnp.float32),
            pltpu.VMEM((D,), jnp.float32),
            pltpu.VMEM((NUMF, BPW), jnp.float32),
            pltpu.VMEM((SUB, 1 + NUMF, D), jnp.float32),
            pltpu.VMEM((BPW,), jnp.int32),
            pltpu.VMEM((BPW, D), jnp.float32),
            pltpu.SemaphoreType.DMA,
        ],
    )
    def sc_kernel(tab_hbm, catT_hbm, num_hbm, w_hbm, bias_hbm, cls_hbm,
                  out_hbm, wv, bv, clsv, numv, buf, idxv, rows, sem):
        wid = lax.axis_index("s") * _NC + lax.axis_index("c")
        base = wid * BPW

        pltpu.sync_copy(w_hbm, wv)
        pltpu.sync_copy(bias_hbm, bv)
        pltpu.sync_copy(cls_hbm.at[0, 0, :], clsv)
        pltpu.sync_copy(num_hbm.at[:, pl.ds(base, BPW)], numv)

        for bl in range(SUB):
            for dd in range(ND):
                sl = pl.ds(dd * _L, _L)
                buf[bl, 0, sl] = clsv[sl]

        def num_body(s, carry):
            for i in range(NUMF):
                row = numv[i, pl.ds(s * SUB, SUB)]
                for bl in range(SUB):
                    x = row[bl]
                    for dd in range(ND):
                        sl = pl.ds(dd * _L, _L)
                        buf[bl, 1 + i, sl] = wv[i, sl] * x + bv[i, sl]
            pltpu.sync_copy(
                buf, out_hbm.at[pl.ds(base + s * SUB, SUB), pl.ds(0, 1 + NUMF), :])
            return carry

        lax.fori_loop(0, NSUB, num_body, 0)

        def cat_body(c, carry):
            pltpu.sync_copy(catT_hbm.at[c, pl.ds(base, BPW)], idxv)
            off = c * V
            for p in range(BPW // _L):
                sl = pl.ds(p * _L, _L)
                idxv[sl] = idxv[sl] + off
            pltpu.async_copy(tab_hbm.at[idxv], rows, sem).wait()
            pltpu.sync_copy(rows, out_hbm.at[pl.ds(base, BPW), 1 + NUMF + c, :])
            return carry

        lax.fori_loop(0, CATF, cat_body, 0)

    return sc_kernel


def kernel(numerical, categorical, W_num, b_num, tables, cls_token):
    B, NUMF = numerical.shape
    CATF = categorical.shape[1]
    V, D = tables.shape[1], tables.shape[2]
    tab_flat = tables.reshape(CATF * V, D)
    cat_t = categorical.T.astype(jnp.int32)
    num_t = numerical.T
    fn = _build(B, NUMF, CATF, V, D)
    return fn(tab_flat, cat_t, num_t, W_num, b_num, cls_token)


# speedup vs baseline: 1.0200x; 1.0200x over previous
import functools

import jax
import jax.numpy as jnp
from jax import lax
from jax.experimental import pallas as pl
from jax.experimental.pallas import tpu as pltpu
from jax.experimental.pallas import tpu_sc as plsc

_NC = 2
_NS = 16
_NW = _NC * _NS
_L = 16
_NBUF = 8


@functools.lru_cache(maxsize=None)
def _build(B, NUMF, CATF, V, D):
    NTOK = 1 + NUMF + CATF
    BPW = B // _NW
    SUB = 16
    NSUB = BPW // SUB
    ND = D // _L
    NBUF = _NBUF

    mesh = plsc.VectorSubcoreMesh(core_axis_name="c", subcore_axis_name="s")

    @functools.partial(
        pl.kernel,
        out_type=jax.ShapeDtypeStruct((B, NTOK, D), jnp.float32),
        mesh=mesh,
        compiler_params=pltpu.CompilerParams(use_tc_tiling_on_sc=False),
        scratch_types=[
            pltpu.VMEM((NUMF, D), jnp.float32),
            pltpu.VMEM((NUMF, D), jnp.float32),
            pltpu.VMEM((D,), jnp.float32),
            pltpu.VMEM((NUMF, BPW), jnp.float32),
            pltpu.VMEM((SUB, 1 + NUMF, D), jnp.float32),
            pltpu.VMEM((CATF, BPW), jnp.int32),
            pltpu.VMEM((NBUF, BPW, D), jnp.float32),
            pltpu.SemaphoreType.DMA,
            pltpu.SemaphoreType.DMA,
        ],
    )
    def sc_kernel(tab_hbm, catT_hbm, numT_hbm, w_hbm, bias_hbm, cls_hbm,
                  out_hbm, wv, bv, clsv, numv, buf, idxm, rows, gsem, wsem):
        wid = lax.axis_index("s") * _NC + lax.axis_index("c")
        base = wid * BPW

        pltpu.sync_copy(catT_hbm.at[:, pl.ds(base, BPW)], idxm)

        def add_offsets(c):
            off = c * V
            for p in range(BPW // _L):
                sl = pl.ds(p * _L, _L)
                idxm[c, sl] = idxm[c, sl] + off

        gathers = {}

        def start_gather(c):
            gathers[c] = pltpu.async_copy(
                tab_hbm.at[idxm.at[c]], rows.at[c % NBUF], gsem)

        for c in range(NBUF - 1):
            add_offsets(c)
        for c in range(NBUF - 1):
            start_gather(c)
        for c in range(NBUF - 1, CATF):
            add_offsets(c)

        pltpu.sync_copy(w_hbm, wv)
        pltpu.sync_copy(bias_hbm, bv)
        pltpu.sync_copy(cls_hbm.at[0, 0, :], clsv)
        pltpu.sync_copy(numT_hbm.at[:, pl.ds(base, BPW)], numv)

        for bl in range(SUB):
            for dd in range(ND):
                sl = pl.ds(dd * _L, _L)
                buf[bl, 0, sl] = clsv[sl]

        def num_body(s, carry):
            for i in range(NUMF):
                row = numv[i, pl.ds(s * SUB, SUB)]
                for bl in range(SUB):
                    x = row[bl]
                    for dd in range(ND):
                        sl = pl.ds(dd * _L, _L)
                        buf[bl, 1 + i, sl] = wv[i, sl] * x + bv[i, sl]
            pltpu.sync_copy(
                buf, out_hbm.at[pl.ds(base + s * SUB, SUB), pl.ds(0, 1 + NUMF), :])
            return carry

        lax.fori_loop(0, NSUB, num_body, 0)

        cat_writes = {}
        waited = set()
        for c in range(CATF):
            j = c + NBUF - 1
            if j < CATF:
                if c > 0:
                    cat_writes[c - 1].wait()
                    waited.add(c - 1)
                start_gather(j)
            gathers[c].wait()
            cat_writes[c] = pltpu.async_copy(
                rows.at[c % NBUF],
                out_hbm.at[pl.ds(base, BPW), 1 + NUMF + c, :], wsem)
        for c in range(CATF):
            if c not in waited:
                cat_writes[c].wait()

    return sc_kernel


def kernel(numerical, categorical, W_num, b_num, tables, cls_token):
    B, NUMF = numerical.shape
    CATF = categorical.shape[1]
    V, D = tables.shape[1], tables.shape[2]
    tab_flat = tables.reshape(CATF * V, D)
    cat_t = categorical.T.astype(jnp.int32)
    num_t = numerical.T
    fn = _build(B, NUMF, CATF, V, D)
    return fn(tab_flat, cat_t, num_t, W_num, b_num, cls_token)
